# baseline (speedup 1.0000x reference)
import functools

import numpy as np
import jax
import jax.numpy as jnp
from jax import lax
from jax.experimental import pallas as pl
from jax.experimental.pallas import tpu as pltpu
from jax.experimental.pallas import tpu_sc as plsc

N = 10000
HID = 128
E = 320000
R2 = 16.0
EDGE_CAP = 131072

NC, NS, L = 2, 16, 16
NW = NC * NS
ROWS = 10112
DUMP = 10000
RSUB = ROWS // NS
B = 32
EPW = 10112
E_PAD = NW * EPW
NCHUNK = EPW // B
CAP_SUB = 16384

NP_ = 10240
WORDS = NP_ // 16
MRB, MCB = 512, 2048
XROWS = NP_ // NW

_LOG1P = (3.5075520e-06, 0.99979246, -0.49697793, 0.31459054,
          -0.18878268, 0.08172681, -0.01720806)

_PACK = ((np.arange(MCB)[:, None] // 16 == np.arange(MCB // 16)[None, :])
         * (1 << (np.arange(MCB) % 16))[:, None]).astype(np.float32)

@functools.lru_cache(maxsize=None)
def _vmesh():
    return plsc.VectorSubcoreMesh(core_axis_name="c", subcore_axis_name="s",
                                  num_cores=NC, num_subcores=NS)


_SC_PARAMS = pltpu.CompilerParams(needs_layout_passes=False)


def _wid():
    return lax.axis_index("s") * NC + lax.axis_index("c")


def _embed_body(x_ref, w1_ref, b1_ref, w2_ref, b2_ref, o_ref):
    t = jnp.maximum(
        jnp.dot(x_ref[...], w1_ref[...], preferred_element_type=jnp.float32)
        + b1_ref[...], 0.0)
    o_ref[...] = (jnp.dot(t, w2_ref[...], preferred_element_type=jnp.float32)
                  + b2_ref[...])


def _embed_tc(x, w1, b1, w2, b2):
    rb = 1264
    return pl.pallas_call(
        _embed_body,
        grid=(ROWS // rb,),
        in_specs=[
            pl.BlockSpec((rb, HID), lambda i: (i, 0)),
            pl.BlockSpec((HID, HID), lambda i: (0, 0)),
            pl.BlockSpec((1, HID), lambda i: (0, 0)),
            pl.BlockSpec((HID, HID), lambda i: (0, 0)),
            pl.BlockSpec((1, HID), lambda i: (0, 0)),
        ],
        out_specs=pl.BlockSpec((rb, HID), lambda i: (i, 0)),
        out_shape=jax.ShapeDtypeStruct((ROWS, HID), jnp.float32),
    )(x, w1, b1, w2, b2)


def _tables_body(h_ref, wd_ref, ws_ref, bd_ref, td_ref, ts_ref):
    h = h_ref[...]
    td_ref[...] = (jnp.dot(h, wd_ref[...], preferred_element_type=jnp.float32)
                   + bd_ref[...])
    ts_ref[...] = jnp.dot(h, ws_ref[...], preferred_element_type=jnp.float32)


def _tables_tc(h, wd, ws, bd):
    rb = 1264
    return pl.pallas_call(
        _tables_body,
        grid=(ROWS // rb,),
        in_specs=[
            pl.BlockSpec((rb, HID), lambda i: (i, 0)),
            pl.BlockSpec((HID, 2 * HID), lambda i: (0, 0)),
            pl.BlockSpec((HID, 2 * HID), lambda i: (0, 0)),
            pl.BlockSpec((1, 2 * HID), lambda i: (0, 0)),
        ],
        out_specs=[pl.BlockSpec((rb, 2 * HID), lambda i: (i, 0))] * 2,
        out_shape=[jax.ShapeDtypeStruct((ROWS, 2 * HID), jnp.float32)] * 2,
    )(h, wd, ws, bd)


def _update_body(h_ref, p0_ref, p1_ref, cnt_ref, o_ref):
    agg = p0_ref[...] + p1_ref[...]
    cnt = jnp.maximum(cnt_ref[...], 1.0)
    o_ref[...] = jnp.maximum(agg / cnt + h_ref[...], 0.0)


def _update_tc(h, p0, p1, cnt):
    rb = 1264
    return pl.pallas_call(
        _update_body,
        grid=(ROWS // rb,),
        in_specs=[
            pl.BlockSpec((rb, HID), lambda i: (i, 0)),
            pl.BlockSpec((rb, HID), lambda i: (i, 0)),
            pl.BlockSpec((rb, HID), lambda i: (i, 0)),
            pl.BlockSpec((rb, 1), lambda i: (i, 0)),
        ],
        out_specs=pl.BlockSpec((rb, HID), lambda i: (i, 0)),
        out_shape=jax.ShapeDtypeStruct((ROWS, HID), jnp.float32),
    )(h, p0, p1, cnt)


def _mask_body(a_ref, bt_ref, pack_ref, o_ref, rs_ref):
    a = a_ref[...]
    bt = bt_ref[...]
    asq = jnp.sum(a * a, axis=1, keepdims=True)
    bsq = jnp.sum(bt * bt, axis=0, keepdims=True)
    d2 = asq + bsq - 2.0 * jnp.dot(a, bt, preferred_element_type=jnp.float32,
                                   precision=lax.Precision.HIGHEST)
    i0 = pl.program_id(0)
    j0 = pl.program_id(1)
    rid = i0 * MRB + lax.broadcasted_iota(jnp.int32, (MRB, MCB), 0)
    cid = j0 * MCB + lax.broadcasted_iota(jnp.int32, (MRB, MCB), 1)
    m = ((d2 < R2) & (rid != cid)).astype(jnp.float32)
    w = jnp.dot(m, pack_ref[...], preferred_element_type=jnp.float32)
    o_ref[...] = w.astype(jnp.int32)

    @pl.when(j0 == 0)
    def _():
        rs_ref[...] = jnp.zeros_like(rs_ref)
    rs_ref[...] += jnp.sum(m, axis=1, keepdims=True)


def _mask_tc(pos_pad, pos_t, pack):
    return pl.pallas_call(
        _mask_body,
        grid=(NP_ // MRB, NP_ // MCB),
        in_specs=[
            pl.BlockSpec((MRB, 8), lambda i, j: (i, 0)),
            pl.BlockSpec((8, MCB), lambda i, j: (0, j)),
            pl.BlockSpec((MCB, MCB // 16), lambda i, j: (0, 0)),
        ],
        out_specs=[pl.BlockSpec((MRB, MCB // 16), lambda i, j: (i, j)),
                   pl.BlockSpec((MRB, 1), lambda i, j: (i, 0))],
        out_shape=[jax.ShapeDtypeStruct((NP_, WORDS), jnp.int32),
                   jax.ShapeDtypeStruct((NP_, 1), jnp.float32)],
    )(pos_pad, pos_t, pack)


@functools.lru_cache(maxsize=None)
def _build_extract():
    return pl.kernel(
        _extract_body,
        out_type=(jax.ShapeDtypeStruct((NW * CAP_SUB,), jnp.int32),
                  jax.ShapeDtypeStruct((NW * CAP_SUB,), jnp.int32),
                  jax.ShapeDtypeStruct((NW * L,), jnp.int32)),
        mesh=_vmesh(),
        compiler_params=_SC_PARAMS,
        scratch_types=[
            pltpu.VMEM((WORDS,), jnp.int32),
            pltpu.VMEM((CAP_SUB,), jnp.int32),
            pltpu.VMEM((CAP_SUB,), jnp.int32),
            pltpu.VMEM((L,), jnp.int32),
        ],
    )


def _extract_sc(maskw_flat):
    return _build_extract()(maskw_flat)


def _extract_body(maskw_hbm, src_hbm, dst_hbm, cnt_hbm,
                  row_v, src_buf, dst_buf, cnt_v):
    w = _wid()

    def fill(k, _):
        src_buf[pl.ds(k * L, L)] = jnp.zeros((L,), jnp.int32)
        dst_buf[pl.ds(k * L, L)] = jnp.full((L,), DUMP, jnp.int32)
        return 0
    lax.fori_loop(0, CAP_SUB // L, fill, 0)

    lane = lax.iota(jnp.int32, L)

    def do_row(i, cnt):
        r = w * XROWS + i
        pltpu.sync_copy(maskw_hbm.at[pl.ds(r * WORDS, WORDS)], row_v)

        def do_group(g, cnt):
            wv = row_v[pl.ds(g * L, L)]
            nz = plsc.all_reduce_population_count(wv != 0)[0]

            def extract(cnt):
                c = cnt
                for b in range(16):
                    mb = ((wv >> b) & 1) != 0
                    jv = (g * L + lane) * 16 + b
                    off = jnp.minimum(c, CAP_SUB - L)
                    plsc.store_compressed(src_buf.at[pl.ds(off, L)], jv,
                                          mask=mb)
                    plsc.store_compressed(dst_buf.at[pl.ds(off, L)],
                                          jnp.full((L,), r, jnp.int32),
                                          mask=mb)
                    pc = plsc.all_reduce_population_count(mb)[0]
                    c = jnp.minimum(c + pc, CAP_SUB - L)
                return c

            return lax.cond(nz > 0, extract, lambda cnt: cnt, cnt)

        return lax.fori_loop(0, WORDS // L, do_group, cnt)

    cnt = lax.fori_loop(0, XROWS, do_row, jnp.int32(0))
    cnt_pad = jnp.maximum(((cnt + (2 * B - 1)) // (2 * B)) * (2 * B),
                          2 * B)

    pltpu.sync_copy(src_buf, src_hbm.at[pl.ds(w * CAP_SUB, CAP_SUB)])
    pltpu.sync_copy(dst_buf, dst_hbm.at[pl.ds(w * CAP_SUB, CAP_SUB)])
    cnt_v[...] = jnp.full((L,), 0, jnp.int32) + cnt_pad
    pltpu.sync_copy(cnt_v, cnt_hbm.at[pl.ds(w * L, L)])


def _sigmoid_softplus(rows_d, rows_s, msg, e):
    for c in range(HID // L):
        zf = rows_d[e, pl.ds(c * L, L)] + rows_s[e, pl.ds(c * L, L)]
        zs = (rows_d[e, pl.ds(HID + c * L, L)]
              + rows_s[e, pl.ds(HID + c * L, L)])
        sig = 1.0 / (1.0 + jnp.exp(-zf))
        t = jnp.exp(-jnp.abs(zs))
        acc = jnp.full((L,), _LOG1P[-1], jnp.float32)
        for a in _LOG1P[-2::-1]:
            acc = acc * t + a
        sp = jnp.maximum(zs, 0.0) + acc
        msg[e, pl.ds(c * L, L)] = sig * sp


def _edge_prologue(zeros_hbm, agg_sh):
    sid = lax.axis_index("s")
    pltpu.sync_copy(zeros_hbm.at[pl.ds(sid * RSUB, RSUB)],
                    agg_sh.at[pl.ds(sid * RSUB, RSUB)])
    plsc.subcore_barrier()


def _edge_epilogue(agg_sh, out_hbm):
    plsc.subcore_barrier()
    cid = lax.axis_index("c")
    sid = lax.axis_index("s")
    base = cid * ROWS + sid * RSUB
    pltpu.sync_copy(agg_sh.at[pl.ds(sid * RSUB, RSUB)],
                    out_hbm.at[pl.ds(base, RSUB)])


def _edge_compute(p, rows_d, rows_s, idx_d, sidx, msg, agg_sh, sem):
    @pl.when(p > 0)
    def _():
        pltpu.make_async_copy(msg, agg_sh.at[sidx], sem).wait()
    for i in range(B // L):
        sidx[pl.ds(i * L, L)] = idx_d[pl.ds(i * L, L)]

    @plsc.parallel_loop(0, B, unroll=8)
    def _(e):
        _sigmoid_softplus(rows_d, rows_s, msg, e)
    pltpu.async_copy(msg, agg_sh.at[sidx], sem, add=True)


def _edge_pipeline(tdst_hbm, tsrc_hbm, srce_hbm, dste_hbm, base0, npair,
                   nch, idx_s0, idx_d0, idx_s1, idx_d1, sidx0, sidx1,
                   rows_d0, rows_s0, rows_d1, rows_s1, msg0, msg1, agg_sh,
                   sem1, sem2, sem3, sem4, sem5, sem6):
    pltpu.sync_copy(srce_hbm.at[pl.ds(base0, B)], idx_s0)
    pltpu.sync_copy(dste_hbm.at[pl.ds(base0, B)], idx_d0)
    cp1 = pltpu.async_copy(tdst_hbm.at[idx_d0], rows_d0, sem1)
    cp2 = pltpu.async_copy(tsrc_hbm.at[idx_s0], rows_s0, sem2)

    def pair(p, _):
        b1 = base0 + (2 * p + 1) * B
        pltpu.sync_copy(srce_hbm.at[pl.ds(b1, B)], idx_s1)
        pltpu.sync_copy(dste_hbm.at[pl.ds(b1, B)], idx_d1)
        cp3 = pltpu.async_copy(tdst_hbm.at[idx_d1], rows_d1, sem3)
        cp4 = pltpu.async_copy(tsrc_hbm.at[idx_s1], rows_s1, sem4)
        cp1.wait()
        cp2.wait()
        _edge_compute(p, rows_d0, rows_s0, idx_d0, sidx0, msg0, agg_sh,
                      sem5)

        @pl.when(2 * p + 2 < nch)
        def _():
            b2 = base0 + (2 * p + 2) * B
            pltpu.sync_copy(srce_hbm.at[pl.ds(b2, B)], idx_s0)
            pltpu.sync_copy(dste_hbm.at[pl.ds(b2, B)], idx_d0)
            pltpu.async_copy(tdst_hbm.at[idx_d0], rows_d0, sem1)
            pltpu.async_copy(tsrc_hbm.at[idx_s0], rows_s0, sem2)

        cp3.wait()
        cp4.wait()
        _edge_compute(p, rows_d1, rows_s1, idx_d1, sidx1, msg1, agg_sh,
                      sem6)
        return 0

    lax.fori_loop(0, npair, pair, 0)
    pltpu.make_async_copy(msg0, agg_sh.at[sidx0], sem5).wait()
    pltpu.make_async_copy(msg1, agg_sh.at[sidx1], sem6).wait()


_EDGE_SCRATCH = [
    pltpu.VMEM((B,), jnp.int32),
    pltpu.VMEM((B,), jnp.int32),
    pltpu.VMEM((B,), jnp.int32),
    pltpu.VMEM((B,), jnp.int32),
    pltpu.VMEM((B,), jnp.int32),
    pltpu.VMEM((B,), jnp.int32),
    pltpu.VMEM((B, 2 * HID), jnp.float32),
    pltpu.VMEM((B, 2 * HID), jnp.float32),
    pltpu.VMEM((B, 2 * HID), jnp.float32),
    pltpu.VMEM((B, 2 * HID), jnp.float32),
    pltpu.VMEM((B, HID), jnp.float32),
    pltpu.VMEM((B, HID), jnp.float32),
    pltpu.VMEM_SHARED((ROWS, HID), jnp.float32),
    pltpu.SemaphoreType.DMA,
    pltpu.SemaphoreType.DMA,
    pltpu.SemaphoreType.DMA,
    pltpu.SemaphoreType.DMA,
    pltpu.SemaphoreType.DMA,
    pltpu.SemaphoreType.DMA,
]

_EDGE_OUT = jax.ShapeDtypeStruct((NC * ROWS, HID), jnp.float32)


@functools.lru_cache(maxsize=None)
def _build_cnt_bond():
    return pl.kernel(
        _cnt_bond_body, out_type=_EDGE_OUT, mesh=_vmesh(),
        compiler_params=_SC_PARAMS,
        scratch_types=[
            pltpu.VMEM((B,), jnp.int32),
            pltpu.VMEM((B, HID), jnp.float32),
            pltpu.VMEM_SHARED((ROWS, HID), jnp.float32),
        ])


def _cnt_bond(dst_b, zeros128):
    return _build_cnt_bond()(dst_b, zeros128)


def _cnt_bond_body(dste_hbm, zeros_hbm, out_hbm, idx_d, ones, cnt_sh):
    w = _wid()
    _edge_prologue(zeros_hbm, cnt_sh)

    def fill(e, _):
        for c in range(HID // L):
            ones[e, pl.ds(c * L, L)] = jnp.full((L,), 1.0, jnp.float32)
        return 0
    lax.fori_loop(0, B, fill, 0)

    def chunk(k, _):
        base = w * EPW + k * B
        pltpu.sync_copy(dste_hbm.at[pl.ds(base, B)], idx_d)
        pltpu.sync_copy(ones, cnt_sh.at[idx_d], add=True)
        return 0
    lax.fori_loop(0, NCHUNK, chunk, 0)
    _edge_epilogue(cnt_sh, out_hbm)


@functools.lru_cache(maxsize=None)
def _build_edges_bond():
    return pl.kernel(_edges_bond_body, out_type=_EDGE_OUT, mesh=_vmesh(),
                     compiler_params=_SC_PARAMS,
                     scratch_types=_EDGE_SCRATCH)


def _edges_bond(td, ts, src_b, dst_b, zeros144):
    return _build_edges_bond()(td, ts, src_b, dst_b, zeros144)


def _edges_bond_body(tdst_hbm, tsrc_hbm, srce_hbm, dste_hbm, zeros_hbm,
                     out_hbm, idx_s0, idx_d0, idx_s1, idx_d1, sidx0,
                     sidx1, rows_d0, rows_s0, rows_d1, rows_s1, msg0,
                     msg1, agg_sh, sem1, sem2, sem3, sem4, sem5, sem6):
    w = _wid()
    _edge_prologue(zeros_hbm, agg_sh)
    _edge_pipeline(tdst_hbm, tsrc_hbm, srce_hbm, dste_hbm, w * EPW,
                   NCHUNK // 2, NCHUNK, idx_s0, idx_d0, idx_s1, idx_d1,
                   sidx0, sidx1, rows_d0, rows_s0, rows_d1, rows_s1,
                   msg0, msg1, agg_sh, sem1, sem2, sem3, sem4, sem5, sem6)
    _edge_epilogue(agg_sh, out_hbm)


@functools.lru_cache(maxsize=None)
def _build_edges_rad():
    return pl.kernel(_edges_rad_body, out_type=_EDGE_OUT, mesh=_vmesh(),
                     compiler_params=_SC_PARAMS,
                     scratch_types=_EDGE_SCRATCH
                     + [pltpu.VMEM((L,), jnp.int32)])


def _edges_rad(td, ts, src_r, dst_r, cnt_r, zeros144):
    return _build_edges_rad()(td, ts, src_r, dst_r, cnt_r, zeros144)


def _edges_rad_body(tdst_hbm, tsrc_hbm, srce_hbm, dste_hbm, cnt_hbm,
                    zeros_hbm, out_hbm, idx_s0, idx_d0, idx_s1, idx_d1,
                    sidx0, sidx1, rows_d0, rows_s0, rows_d1, rows_s1,
                    msg0, msg1, agg_sh, sem1, sem2, sem3, sem4, sem5,
                    sem6, cnt_v):
    w = _wid()
    _edge_prologue(zeros_hbm, agg_sh)
    pltpu.sync_copy(cnt_hbm.at[pl.ds(w * L, L)], cnt_v)
    nch = cnt_v[...][0] // B
    _edge_pipeline(tdst_hbm, tsrc_hbm, srce_hbm, dste_hbm, w * CAP_SUB,
                   nch // 2, nch, idx_s0, idx_d0, idx_s1, idx_d1,
                   sidx0, sidx1, rows_d0, rows_s0, rows_d1, rows_s1,
                   msg0, msg1, agg_sh, sem1, sem2, sem3, sem4, sem5, sem6)
    _edge_epilogue(agg_sh, out_hbm)


def kernel(x_prot, v_prot, edge_index, embed_W1, embed_b1, embed_W2, embed_b2,
           bond1_Wf, bond1_bf, bond1_Ws, bond1_bs,
           bond2_Wf, bond2_bf, bond2_Ws, bond2_bs,
           radi1_Wf, radi1_bf, radi1_Ws, radi1_bs,
           radi2_Wf, radi2_bf, radi2_Ws, radi2_bs):
    f32 = jnp.float32
    x_pad = jnp.zeros((ROWS, HID), f32).at[:N].set(x_prot)
    src_b = jnp.concatenate(
        [edge_index[0], jnp.zeros((E_PAD - E,), jnp.int32)])
    dst_b = jnp.concatenate(
        [edge_index[1], jnp.full((E_PAD - E,), DUMP, jnp.int32)])

    pad = jnp.arange(NP_) >= N
    pos = jnp.zeros((NP_, 8), f32).at[:N, :3].set(v_prot.astype(f32) - 50.0)
    big = (100.0 + 10.0 * (jnp.arange(NP_) - N)).astype(f32)
    pos = pos.at[:, 0].set(jnp.where(pad, big, pos[:, 0]))
    maskw, rowsum = _mask_tc(pos, pos.T, jnp.asarray(_PACK))
    maskw_flat = maskw.reshape(-1)
    src_r, dst_r, cnt_r = _extract_sc(maskw_flat)

    zeros128 = jnp.zeros((ROWS, HID), f32)
    cb = _cnt_bond(dst_b, zeros128)
    cnt_bond_col = (cb[:ROWS, :1] + cb[ROWS:, :1])
    cnt_rad_col = rowsum[:ROWS]
    h = _embed_tc(x_pad, embed_W1, embed_b1.reshape(1, HID),
                  embed_W2, embed_b2.reshape(1, HID))

    layers = [
        (bond1_Wf, bond1_bf, bond1_Ws, bond1_bs, True),
        (bond2_Wf, bond2_bf, bond2_Ws, bond2_bs, True),
        (radi1_Wf, radi1_bf, radi1_Ws, radi1_bs, False),
        (radi2_Wf, radi2_bf, radi2_Ws, radi2_bs, False),
    ]
    for Wf, bf, Ws, bs, is_bond in layers:
        wd = jnp.concatenate([Wf[:HID], Ws[:HID]], axis=1)
        wsrc = jnp.concatenate([Wf[HID:], Ws[HID:]], axis=1)
        bd = jnp.concatenate([bf, bs]).reshape(1, 2 * HID)
        td, ts = _tables_tc(h, wd, wsrc, bd)
        if is_bond:
            parts = _edges_bond(td, ts, src_b, dst_b, zeros128)
            cnt_col = cnt_bond_col
        else:
            parts = _edges_rad(td, ts, src_r, dst_r, cnt_r, zeros128)
            cnt_col = cnt_rad_col
        h = _update_tc(h, parts[:ROWS], parts[ROWS:], cnt_col)
    return h[:N]


# speedup vs baseline: 3.8928x; 3.8928x over previous
import functools

import numpy as np
import jax
import jax.numpy as jnp
from jax import lax
from jax.experimental import pallas as pl
from jax.experimental.pallas import tpu as pltpu
from jax.experimental.pallas import tpu_sc as plsc

N = 10000
HID = 128
E = 320000
R2 = 16.0
EDGE_CAP = 131072

NC, NS, L = 2, 16, 16
NW = NC * NS
ROWS = 10112
DUMP = 10000
RSUB = ROWS // NS
B = 32
EPW = 10112
E_PAD = NW * EPW
NCHUNK = EPW // B
CAP_SUB = 16384

NP_ = 10240
WORDS = NP_ // 16
MRB, MCB = 512, 2048
XROWS = NP_ // NW

_LOG1P = (3.5075520e-06, 0.99979246, -0.49697793, 0.31459054,
          -0.18878268, 0.08172681, -0.01720806)

_PACK = ((np.arange(MCB)[:, None] // 16 == np.arange(MCB // 16)[None, :])
         * (1 << (np.arange(MCB) % 16))[:, None]).astype(np.float32)

@functools.lru_cache(maxsize=None)
def _vmesh():
    return plsc.VectorSubcoreMesh(core_axis_name="c", subcore_axis_name="s",
                                  num_cores=NC, num_subcores=NS)


_SC_PARAMS = pltpu.CompilerParams(needs_layout_passes=False)


def _wid():
    return lax.axis_index("s") * NC + lax.axis_index("c")


def _embed_body(x_ref, w1_ref, b1_ref, w2_ref, b2_ref, o_ref):
    t = jnp.maximum(
        jnp.dot(x_ref[...], w1_ref[...], preferred_element_type=jnp.float32)
        + b1_ref[...], 0.0)
    o_ref[...] = (jnp.dot(t, w2_ref[...], preferred_element_type=jnp.float32)
                  + b2_ref[...])


def _embed_tc(x, w1, b1, w2, b2):
    rb = 1264
    return pl.pallas_call(
        _embed_body,
        grid=(ROWS // rb,),
        in_specs=[
            pl.BlockSpec((rb, HID), lambda i: (i, 0)),
            pl.BlockSpec((HID, HID), lambda i: (0, 0)),
            pl.BlockSpec((1, HID), lambda i: (0, 0)),
            pl.BlockSpec((HID, HID), lambda i: (0, 0)),
            pl.BlockSpec((1, HID), lambda i: (0, 0)),
        ],
        out_specs=pl.BlockSpec((rb, HID), lambda i: (i, 0)),
        out_shape=jax.ShapeDtypeStruct((ROWS, HID), jnp.float32),
    )(x, w1, b1, w2, b2)


def _tables_body(h_ref, wd_ref, ws_ref, bd_ref, td_ref, ts_ref):
    h = h_ref[...]
    td_ref[...] = (jnp.dot(h, wd_ref[...], preferred_element_type=jnp.float32)
                   + bd_ref[...])
    ts_ref[...] = jnp.dot(h, ws_ref[...], preferred_element_type=jnp.float32)


def _tables_tc(h, wd, ws, bd):
    rb = 1264
    return pl.pallas_call(
        _tables_body,
        grid=(ROWS // rb,),
        in_specs=[
            pl.BlockSpec((rb, HID), lambda i: (i, 0)),
            pl.BlockSpec((HID, 2 * HID), lambda i: (0, 0)),
            pl.BlockSpec((HID, 2 * HID), lambda i: (0, 0)),
            pl.BlockSpec((1, 2 * HID), lambda i: (0, 0)),
        ],
        out_specs=[pl.BlockSpec((rb, 2 * HID), lambda i: (i, 0))] * 2,
        out_shape=[jax.ShapeDtypeStruct((ROWS, 2 * HID), jnp.float32)] * 2,
    )(h, wd, ws, bd)


def _update_body(h_ref, p0_ref, p1_ref, cnt_ref, o_ref):
    agg = p0_ref[...] + p1_ref[...]
    cnt = jnp.maximum(cnt_ref[...], 1.0)
    o_ref[...] = jnp.maximum(agg / cnt + h_ref[...], 0.0)


def _update_tc(h, p0, p1, cnt):
    rb = 1264
    return pl.pallas_call(
        _update_body,
        grid=(ROWS // rb,),
        in_specs=[
            pl.BlockSpec((rb, HID), lambda i: (i, 0)),
            pl.BlockSpec((rb, HID), lambda i: (i, 0)),
            pl.BlockSpec((rb, HID), lambda i: (i, 0)),
            pl.BlockSpec((rb, 1), lambda i: (i, 0)),
        ],
        out_specs=pl.BlockSpec((rb, HID), lambda i: (i, 0)),
        out_shape=jax.ShapeDtypeStruct((ROWS, HID), jnp.float32),
    )(h, p0, p1, cnt)


def _mask_body(a_ref, bt_ref, pack_ref, o_ref, rs_ref):
    a = a_ref[...]
    bt = bt_ref[...]
    asq = jnp.sum(a * a, axis=1, keepdims=True)
    bsq = jnp.sum(bt * bt, axis=0, keepdims=True)
    d2 = asq + bsq - 2.0 * jnp.dot(a, bt, preferred_element_type=jnp.float32,
                                   precision=lax.Precision.HIGHEST)
    i0 = pl.program_id(0)
    j0 = pl.program_id(1)
    rid = i0 * MRB + lax.broadcasted_iota(jnp.int32, (MRB, MCB), 0)
    cid = j0 * MCB + lax.broadcasted_iota(jnp.int32, (MRB, MCB), 1)
    m = ((d2 < R2) & (rid != cid)).astype(jnp.float32)
    w = jnp.dot(m, pack_ref[...], preferred_element_type=jnp.float32)
    o_ref[...] = w.astype(jnp.int32)

    @pl.when(j0 == 0)
    def _():
        rs_ref[...] = jnp.zeros_like(rs_ref)
    rs_ref[...] += jnp.sum(m, axis=1, keepdims=True)


def _mask_tc(pos_pad, pos_t, pack):
    return pl.pallas_call(
        _mask_body,
        grid=(NP_ // MRB, NP_ // MCB),
        in_specs=[
            pl.BlockSpec((MRB, 8), lambda i, j: (i, 0)),
            pl.BlockSpec((8, MCB), lambda i, j: (0, j)),
            pl.BlockSpec((MCB, MCB // 16), lambda i, j: (0, 0)),
        ],
        out_specs=[pl.BlockSpec((MRB, MCB // 16), lambda i, j: (i, j)),
                   pl.BlockSpec((MRB, 1), lambda i, j: (i, 0))],
        out_shape=[jax.ShapeDtypeStruct((NP_, WORDS), jnp.int32),
                   jax.ShapeDtypeStruct((NP_, 1), jnp.float32)],
    )(pos_pad, pos_t, pack)


@functools.lru_cache(maxsize=None)
def _build_extract():
    return pl.kernel(
        _extract_body,
        out_type=(jax.ShapeDtypeStruct((NW * CAP_SUB,), jnp.int32),
                  jax.ShapeDtypeStruct((NW * CAP_SUB,), jnp.int32),
                  jax.ShapeDtypeStruct((NW * L,), jnp.int32)),
        mesh=_vmesh(),
        compiler_params=_SC_PARAMS,
        scratch_types=[
            pltpu.VMEM((WORDS,), jnp.int32),
            pltpu.VMEM((CAP_SUB,), jnp.int32),
            pltpu.VMEM((CAP_SUB,), jnp.int32),
            pltpu.VMEM((L,), jnp.int32),
        ],
    )


def _extract_sc(maskw_flat):
    return _build_extract()(maskw_flat)


def _extract_body(maskw_hbm, src_hbm, dst_hbm, cnt_hbm,
                  row_v, src_buf, dst_buf, cnt_v):
    w = _wid()

    def fill(k, _):
        src_buf[pl.ds(k * L, L)] = jnp.zeros((L,), jnp.int32)
        dst_buf[pl.ds(k * L, L)] = jnp.full((L,), DUMP, jnp.int32)
        return 0
    lax.fori_loop(0, CAP_SUB // L, fill, 0)

    lane = lax.iota(jnp.int32, L)

    def do_row(i, cnt):
        r = w * XROWS + i
        pltpu.sync_copy(maskw_hbm.at[pl.ds(r * WORDS, WORDS)], row_v)

        def do_group(g, cnt):
            wv = row_v[pl.ds(g * L, L)]
            nz = plsc.all_reduce_population_count(wv != 0)[0]

            def extract(cnt):
                c = cnt
                for b in range(16):
                    mb = ((wv >> b) & 1) != 0
                    jv = (g * L + lane) * 16 + b
                    off = jnp.minimum(c, CAP_SUB - L)
                    plsc.store_compressed(src_buf.at[pl.ds(off, L)], jv,
                                          mask=mb)
                    plsc.store_compressed(dst_buf.at[pl.ds(off, L)],
                                          jnp.full((L,), r, jnp.int32),
                                          mask=mb)
                    pc = plsc.all_reduce_population_count(mb)[0]
                    c = jnp.minimum(c + pc, CAP_SUB - L)
                return c

            return lax.cond(nz > 0, extract, lambda cnt: cnt, cnt)

        return lax.fori_loop(0, WORDS // L, do_group, cnt)

    cnt = lax.fori_loop(0, XROWS, do_row, jnp.int32(0))
    cnt_pad = jnp.maximum(((cnt + (2 * B - 1)) // (2 * B)) * (2 * B),
                          2 * B)

    pltpu.sync_copy(src_buf, src_hbm.at[pl.ds(w * CAP_SUB, CAP_SUB)])
    pltpu.sync_copy(dst_buf, dst_hbm.at[pl.ds(w * CAP_SUB, CAP_SUB)])
    cnt_v[...] = jnp.full((L,), 0, jnp.int32) + cnt_pad
    pltpu.sync_copy(cnt_v, cnt_hbm.at[pl.ds(w * L, L)])


def _sigmoid_softplus(rows_d, rows_s, msg, e):
    for c in range(HID // L):
        zf = rows_d[e, pl.ds(c * L, L)] + rows_s[e, pl.ds(c * L, L)]
        zs = (rows_d[e, pl.ds(HID + c * L, L)]
              + rows_s[e, pl.ds(HID + c * L, L)])
        sig = 1.0 / (1.0 + jnp.exp(-zf))
        t = jnp.exp(-jnp.abs(zs))
        acc = jnp.full((L,), _LOG1P[-1], jnp.float32)
        for a in _LOG1P[-2::-1]:
            acc = acc * t + a
        sp = jnp.maximum(zs, 0.0) + acc
        msg[e, pl.ds(c * L, L)] = sig * sp


def _edge_prologue(zeros_hbm, agg_sh):
    sid = lax.axis_index("s")
    pltpu.sync_copy(zeros_hbm.at[pl.ds(sid * RSUB, RSUB)],
                    agg_sh.at[pl.ds(sid * RSUB, RSUB)])
    plsc.subcore_barrier()


def _edge_epilogue(agg_sh, out_hbm):
    plsc.subcore_barrier()
    cid = lax.axis_index("c")
    sid = lax.axis_index("s")
    base = cid * ROWS + sid * RSUB
    pltpu.sync_copy(agg_sh.at[pl.ds(sid * RSUB, RSUB)],
                    out_hbm.at[pl.ds(base, RSUB)])


def _edge_compute(p, rows_d, rows_s, idx_d, sidx, msg, agg_sh, sem):
    @pl.when(p > 0)
    def _():
        pltpu.make_async_copy(msg, agg_sh.at[sidx], sem).wait()
    for i in range(B // L):
        sidx[pl.ds(i * L, L)] = idx_d[pl.ds(i * L, L)]

    @plsc.parallel_loop(0, B, unroll=2)
    def _(e):
        _sigmoid_softplus(rows_d, rows_s, msg, e)
    pltpu.async_copy(msg, agg_sh.at[sidx], sem, add=True)


def _edge_pipeline(tdst_hbm, tsrc_hbm, srce_hbm, dste_hbm, base0, npair,
                   nch, idx_s0, idx_d0, idx_s1, idx_d1, sidx0, sidx1,
                   rows_d0, rows_s0, rows_d1, rows_s1, msg0, msg1, agg_sh,
                   sem1, sem2, sem3, sem4, sem5, sem6):
    pltpu.sync_copy(srce_hbm.at[pl.ds(base0, B)], idx_s0)
    pltpu.sync_copy(dste_hbm.at[pl.ds(base0, B)], idx_d0)
    cp1 = pltpu.async_copy(tdst_hbm.at[idx_d0], rows_d0, sem1)
    cp2 = pltpu.async_copy(tsrc_hbm.at[idx_s0], rows_s0, sem2)

    def pair(p, _):
        b1 = base0 + (2 * p + 1) * B
        pltpu.sync_copy(srce_hbm.at[pl.ds(b1, B)], idx_s1)
        pltpu.sync_copy(dste_hbm.at[pl.ds(b1, B)], idx_d1)
        cp3 = pltpu.async_copy(tdst_hbm.at[idx_d1], rows_d1, sem3)
        cp4 = pltpu.async_copy(tsrc_hbm.at[idx_s1], rows_s1, sem4)
        cp1.wait()
        cp2.wait()
        _edge_compute(p, rows_d0, rows_s0, idx_d0, sidx0, msg0, agg_sh,
                      sem5)

        @pl.when(2 * p + 2 < nch)
        def _():
            b2 = base0 + (2 * p + 2) * B
            pltpu.sync_copy(srce_hbm.at[pl.ds(b2, B)], idx_s0)
            pltpu.sync_copy(dste_hbm.at[pl.ds(b2, B)], idx_d0)
            pltpu.async_copy(tdst_hbm.at[idx_d0], rows_d0, sem1)
            pltpu.async_copy(tsrc_hbm.at[idx_s0], rows_s0, sem2)

        cp3.wait()
        cp4.wait()
        _edge_compute(p, rows_d1, rows_s1, idx_d1, sidx1, msg1, agg_sh,
                      sem6)
        return 0

    lax.fori_loop(0, npair, pair, 0)
    pltpu.make_async_copy(msg0, agg_sh.at[sidx0], sem5).wait()
    pltpu.make_async_copy(msg1, agg_sh.at[sidx1], sem6).wait()


_EDGE_SCRATCH = [
    pltpu.VMEM((B,), jnp.int32),
    pltpu.VMEM((B,), jnp.int32),
    pltpu.VMEM((B,), jnp.int32),
    pltpu.VMEM((B,), jnp.int32),
    pltpu.VMEM((B,), jnp.int32),
    pltpu.VMEM((B,), jnp.int32),
    pltpu.VMEM((B, 2 * HID), jnp.float32),
    pltpu.VMEM((B, 2 * HID), jnp.float32),
    pltpu.VMEM((B, 2 * HID), jnp.float32),
    pltpu.VMEM((B, 2 * HID), jnp.float32),
    pltpu.VMEM((B, HID), jnp.float32),
    pltpu.VMEM((B, HID), jnp.float32),
    pltpu.VMEM_SHARED((ROWS, HID), jnp.float32),
    pltpu.SemaphoreType.DMA,
    pltpu.SemaphoreType.DMA,
    pltpu.SemaphoreType.DMA,
    pltpu.SemaphoreType.DMA,
    pltpu.SemaphoreType.DMA,
    pltpu.SemaphoreType.DMA,
]

_EDGE_OUT = jax.ShapeDtypeStruct((NC * ROWS, HID), jnp.float32)


@functools.lru_cache(maxsize=None)
def _build_cnt_bond():
    return pl.kernel(
        _cnt_bond_body, out_type=_EDGE_OUT, mesh=_vmesh(),
        compiler_params=_SC_PARAMS,
        scratch_types=[
            pltpu.VMEM((B,), jnp.int32),
            pltpu.VMEM((B, HID), jnp.float32),
            pltpu.VMEM_SHARED((ROWS, HID), jnp.float32),
        ])


def _cnt_bond(dst_b, zeros128):
    return _build_cnt_bond()(dst_b, zeros128)


def _cnt_bond_body(dste_hbm, zeros_hbm, out_hbm, idx_d, ones, cnt_sh):
    w = _wid()
    _edge_prologue(zeros_hbm, cnt_sh)

    def fill(e, _):
        for c in range(HID // L):
            ones[e, pl.ds(c * L, L)] = jnp.full((L,), 1.0, jnp.float32)
        return 0
    lax.fori_loop(0, B, fill, 0)

    def chunk(k, _):
        base = w * EPW + k * B
        pltpu.sync_copy(dste_hbm.at[pl.ds(base, B)], idx_d)
        pltpu.sync_copy(ones, cnt_sh.at[idx_d], add=True)
        return 0
    lax.fori_loop(0, NCHUNK, chunk, 0)
    _edge_epilogue(cnt_sh, out_hbm)


@functools.lru_cache(maxsize=None)
def _build_edges_bond():
    return pl.kernel(_edges_bond_body, out_type=_EDGE_OUT, mesh=_vmesh(),
                     compiler_params=_SC_PARAMS,
                     scratch_types=_EDGE_SCRATCH)


def _edges_bond(td, ts, src_b, dst_b, zeros144):
    return _build_edges_bond()(td, ts, src_b, dst_b, zeros144)


def _edges_bond_body(tdst_hbm, tsrc_hbm, srce_hbm, dste_hbm, zeros_hbm,
                     out_hbm, idx_s0, idx_d0, idx_s1, idx_d1, sidx0,
                     sidx1, rows_d0, rows_s0, rows_d1, rows_s1, msg0,
                     msg1, agg_sh, sem1, sem2, sem3, sem4, sem5, sem6):
    w = _wid()
    _edge_prologue(zeros_hbm, agg_sh)
    _edge_pipeline(tdst_hbm, tsrc_hbm, srce_hbm, dste_hbm, w * EPW,
                   NCHUNK // 2, NCHUNK, idx_s0, idx_d0, idx_s1, idx_d1,
                   sidx0, sidx1, rows_d0, rows_s0, rows_d1, rows_s1,
                   msg0, msg1, agg_sh, sem1, sem2, sem3, sem4, sem5, sem6)
    _edge_epilogue(agg_sh, out_hbm)


@functools.lru_cache(maxsize=None)
def _build_edges_rad():
    return pl.kernel(_edges_rad_body, out_type=_EDGE_OUT, mesh=_vmesh(),
                     compiler_params=_SC_PARAMS,
                     scratch_types=_EDGE_SCRATCH
                     + [pltpu.VMEM((L,), jnp.int32)])


def _edges_rad(td, ts, src_r, dst_r, cnt_r, zeros144):
    return _build_edges_rad()(td, ts, src_r, dst_r, cnt_r, zeros144)


def _edges_rad_body(tdst_hbm, tsrc_hbm, srce_hbm, dste_hbm, cnt_hbm,
                    zeros_hbm, out_hbm, idx_s0, idx_d0, idx_s1, idx_d1,
                    sidx0, sidx1, rows_d0, rows_s0, rows_d1, rows_s1,
                    msg0, msg1, agg_sh, sem1, sem2, sem3, sem4, sem5,
                    sem6, cnt_v):
    w = _wid()
    _edge_prologue(zeros_hbm, agg_sh)
    pltpu.sync_copy(cnt_hbm.at[pl.ds(w * L, L)], cnt_v)
    nch = cnt_v[...][0] // B
    _edge_pipeline(tdst_hbm, tsrc_hbm, srce_hbm, dste_hbm, w * CAP_SUB,
                   nch // 2, nch, idx_s0, idx_d0, idx_s1, idx_d1,
                   sidx0, sidx1, rows_d0, rows_s0, rows_d1, rows_s1,
                   msg0, msg1, agg_sh, sem1, sem2, sem3, sem4, sem5, sem6)
    _edge_epilogue(agg_sh, out_hbm)


def kernel(x_prot, v_prot, edge_index, embed_W1, embed_b1, embed_W2, embed_b2,
           bond1_Wf, bond1_bf, bond1_Ws, bond1_bs,
           bond2_Wf, bond2_bf, bond2_Ws, bond2_bs,
           radi1_Wf, radi1_bf, radi1_Ws, radi1_bs,
           radi2_Wf, radi2_bf, radi2_Ws, radi2_bs):
    f32 = jnp.float32
    x_pad = jnp.zeros((ROWS, HID), f32).at[:N].set(x_prot)
    src_b = jnp.concatenate(
        [edge_index[0], jnp.zeros((E_PAD - E,), jnp.int32)])
    dst_b = jnp.concatenate(
        [edge_index[1], jnp.full((E_PAD - E,), DUMP, jnp.int32)])

    pad = jnp.arange(NP_) >= N
    pos = jnp.zeros((NP_, 8), f32).at[:N, :3].set(v_prot.astype(f32) - 50.0)
    big = (100.0 + 10.0 * (jnp.arange(NP_) - N)).astype(f32)
    pos = pos.at[:, 0].set(jnp.where(pad, big, pos[:, 0]))
    maskw, rowsum = _mask_tc(pos, pos.T, jnp.asarray(_PACK))
    maskw_flat = maskw.reshape(-1)
    src_r, dst_r, cnt_r = _extract_sc(maskw_flat)

    zeros128 = jnp.zeros((ROWS, HID), f32)
    cb = _cnt_bond(dst_b, zeros128)
    cnt_bond_col = (cb[:ROWS, :1] + cb[ROWS:, :1])
    cnt_rad_col = rowsum[:ROWS]
    h = _embed_tc(x_pad, embed_W1, embed_b1.reshape(1, HID),
                  embed_W2, embed_b2.reshape(1, HID))

    layers = [
        (bond1_Wf, bond1_bf, bond1_Ws, bond1_bs, True),
        (bond2_Wf, bond2_bf, bond2_Ws, bond2_bs, True),
        (radi1_Wf, radi1_bf, radi1_Ws, radi1_bs, False),
        (radi2_Wf, radi2_bf, radi2_Ws, radi2_bs, False),
    ]
    for Wf, bf, Ws, bs, is_bond in layers:
        wd = jnp.concatenate([Wf[:HID], Ws[:HID]], axis=1)
        wsrc = jnp.concatenate([Wf[HID:], Ws[HID:]], axis=1)
        bd = jnp.concatenate([bf, bs]).reshape(1, 2 * HID)
        td, ts = _tables_tc(h, wd, wsrc, bd)
        if is_bond:
            parts = _edges_bond(td, ts, src_b, dst_b, zeros128)
            cnt_col = cnt_bond_col
        else:
            parts = _edges_rad(td, ts, src_r, dst_r, cnt_r, zeros128)
            cnt_col = cnt_rad_col
        h = _update_tc(h, parts[:ROWS], parts[ROWS:], cnt_col)
    return h[:N]


# speedup vs baseline: 4.6056x; 1.1831x over previous
import functools

import numpy as np
import jax
import jax.numpy as jnp
from jax import lax
from jax.experimental import pallas as pl
from jax.experimental.pallas import tpu as pltpu
from jax.experimental.pallas import tpu_sc as plsc

N = 10000
HID = 128
E = 320000
R2 = 16.0
EDGE_CAP = 131072

NC, NS, L = 2, 16, 16
NW = NC * NS
ROWS = 10112
DUMP = 10000
RSUB = ROWS // NS
B = 32
EPW = 10112
E_PAD = NW * EPW
NCHUNK = EPW // B
CAP_SUB = 16384

NP_ = 10240
WORDS = NP_ // 16
MRB, MCB = 512, 2048
XROWS = NP_ // NW

_LOG1P = (3.5075520e-06, 0.99979246, -0.49697793, 0.31459054,
          -0.18878268, 0.08172681, -0.01720806)

_PACK = ((np.arange(MCB)[:, None] // 16 == np.arange(MCB // 16)[None, :])
         * (1 << (np.arange(MCB) % 16))[:, None]).astype(np.float32)

@functools.lru_cache(maxsize=None)
def _vmesh():
    return plsc.VectorSubcoreMesh(core_axis_name="c", subcore_axis_name="s",
                                  num_cores=NC, num_subcores=NS)


_SC_PARAMS = pltpu.CompilerParams(needs_layout_passes=False)


def _wid():
    return lax.axis_index("s") * NC + lax.axis_index("c")


def _embed_body(x_ref, w1_ref, b1_ref, w2_ref, b2_ref, o_ref):
    t = jnp.maximum(
        jnp.dot(x_ref[...], w1_ref[...], preferred_element_type=jnp.float32)
        + b1_ref[...], 0.0)
    o_ref[...] = (jnp.dot(t, w2_ref[...], preferred_element_type=jnp.float32)
                  + b2_ref[...])


def _embed_tc(x, w1, b1, w2, b2):
    rb = 1264
    return pl.pallas_call(
        _embed_body,
        grid=(ROWS // rb,),
        in_specs=[
            pl.BlockSpec((rb, HID), lambda i: (i, 0)),
            pl.BlockSpec((HID, HID), lambda i: (0, 0)),
            pl.BlockSpec((1, HID), lambda i: (0, 0)),
            pl.BlockSpec((HID, HID), lambda i: (0, 0)),
            pl.BlockSpec((1, HID), lambda i: (0, 0)),
        ],
        out_specs=pl.BlockSpec((rb, HID), lambda i: (i, 0)),
        out_shape=jax.ShapeDtypeStruct((ROWS, HID), jnp.float32),
    )(x, w1, b1, w2, b2)


def _tables_body(h_ref, wd_ref, ws_ref, bd_ref, td_ref, ts_ref):
    h = h_ref[...]
    td_ref[...] = (jnp.dot(h, wd_ref[...], preferred_element_type=jnp.float32)
                   + bd_ref[...])
    ts_ref[...] = jnp.dot(h, ws_ref[...], preferred_element_type=jnp.float32)


def _tables_tc(h, wd, ws, bd):
    rb = 1264
    return pl.pallas_call(
        _tables_body,
        grid=(ROWS // rb,),
        in_specs=[
            pl.BlockSpec((rb, HID), lambda i: (i, 0)),
            pl.BlockSpec((HID, 2 * HID), lambda i: (0, 0)),
            pl.BlockSpec((HID, 2 * HID), lambda i: (0, 0)),
            pl.BlockSpec((1, 2 * HID), lambda i: (0, 0)),
        ],
        out_specs=[pl.BlockSpec((rb, 2 * HID), lambda i: (i, 0))] * 2,
        out_shape=[jax.ShapeDtypeStruct((ROWS, 2 * HID), jnp.float32)] * 2,
    )(h, wd, ws, bd)


def _update_body(h_ref, p0_ref, p1_ref, cnt_ref, o_ref):
    agg = p0_ref[...] + p1_ref[...]
    cnt = jnp.maximum(cnt_ref[...], 1.0)
    o_ref[...] = jnp.maximum(agg / cnt + h_ref[...], 0.0)


def _update_tc(h, p0, p1, cnt):
    rb = 1264
    return pl.pallas_call(
        _update_body,
        grid=(ROWS // rb,),
        in_specs=[
            pl.BlockSpec((rb, HID), lambda i: (i, 0)),
            pl.BlockSpec((rb, HID), lambda i: (i, 0)),
            pl.BlockSpec((rb, HID), lambda i: (i, 0)),
            pl.BlockSpec((rb, 1), lambda i: (i, 0)),
        ],
        out_specs=pl.BlockSpec((rb, HID), lambda i: (i, 0)),
        out_shape=jax.ShapeDtypeStruct((ROWS, HID), jnp.float32),
    )(h, p0, p1, cnt)


def _mask_body(a_ref, bt_ref, pack_ref, o_ref, rs_ref):
    a = a_ref[...]
    bt = bt_ref[...]
    asq = jnp.sum(a * a, axis=1, keepdims=True)
    bsq = jnp.sum(bt * bt, axis=0, keepdims=True)
    d2 = asq + bsq - 2.0 * jnp.dot(a, bt, preferred_element_type=jnp.float32,
                                   precision=lax.Precision.HIGHEST)
    i0 = pl.program_id(0)
    j0 = pl.program_id(1)
    rid = i0 * MRB + lax.broadcasted_iota(jnp.int32, (MRB, MCB), 0)
    cid = j0 * MCB + lax.broadcasted_iota(jnp.int32, (MRB, MCB), 1)
    m = ((d2 < R2) & (rid != cid)).astype(jnp.float32)
    w = jnp.dot(m, pack_ref[...], preferred_element_type=jnp.float32)
    o_ref[...] = w.astype(jnp.int32)

    @pl.when(j0 == 0)
    def _():
        rs_ref[...] = jnp.zeros_like(rs_ref)
    rs_ref[...] += jnp.sum(m, axis=1, keepdims=True)


def _mask_tc(pos_pad, pos_t, pack):
    return pl.pallas_call(
        _mask_body,
        grid=(NP_ // MRB, NP_ // MCB),
        in_specs=[
            pl.BlockSpec((MRB, 8), lambda i, j: (i, 0)),
            pl.BlockSpec((8, MCB), lambda i, j: (0, j)),
            pl.BlockSpec((MCB, MCB // 16), lambda i, j: (0, 0)),
        ],
        out_specs=[pl.BlockSpec((MRB, MCB // 16), lambda i, j: (i, j)),
                   pl.BlockSpec((MRB, 1), lambda i, j: (i, 0))],
        out_shape=[jax.ShapeDtypeStruct((NP_, WORDS), jnp.int32),
                   jax.ShapeDtypeStruct((NP_, 1), jnp.float32)],
    )(pos_pad, pos_t, pack)


@functools.lru_cache(maxsize=None)
def _build_extract():
    return pl.kernel(
        _extract_body,
        out_type=(jax.ShapeDtypeStruct((NW * CAP_SUB,), jnp.int32),
                  jax.ShapeDtypeStruct((NW * CAP_SUB,), jnp.int32),
                  jax.ShapeDtypeStruct((NW * L,), jnp.int32)),
        mesh=_vmesh(),
        compiler_params=_SC_PARAMS,
        scratch_types=[
            pltpu.VMEM((WORDS,), jnp.int32),
            pltpu.VMEM((CAP_SUB,), jnp.int32),
            pltpu.VMEM((CAP_SUB,), jnp.int32),
            pltpu.VMEM((L,), jnp.int32),
        ],
    )


def _extract_sc(maskw_flat):
    return _build_extract()(maskw_flat)


def _extract_body(maskw_hbm, src_hbm, dst_hbm, cnt_hbm,
                  row_v, src_buf, dst_buf, cnt_v):
    w = _wid()

    def fill(k, _):
        src_buf[pl.ds(k * L, L)] = jnp.zeros((L,), jnp.int32)
        dst_buf[pl.ds(k * L, L)] = jnp.full((L,), DUMP, jnp.int32)
        return 0
    lax.fori_loop(0, CAP_SUB // L, fill, 0)

    lane = lax.iota(jnp.int32, L)

    def do_row(i, cnt):
        r = w * XROWS + i
        pltpu.sync_copy(maskw_hbm.at[pl.ds(r * WORDS, WORDS)], row_v)

        def do_group(g, cnt):
            wv = row_v[pl.ds(g * L, L)]
            nz = plsc.all_reduce_population_count(wv != 0)[0]

            def extract(cnt):
                c = cnt
                for b in range(16):
                    mb = ((wv >> b) & 1) != 0
                    jv = (g * L + lane) * 16 + b
                    off = jnp.minimum(c, CAP_SUB - L)
                    plsc.store_compressed(src_buf.at[pl.ds(off, L)], jv,
                                          mask=mb)
                    plsc.store_compressed(dst_buf.at[pl.ds(off, L)],
                                          jnp.full((L,), r, jnp.int32),
                                          mask=mb)
                    pc = plsc.all_reduce_population_count(mb)[0]
                    c = jnp.minimum(c + pc, CAP_SUB - L)
                return c

            return lax.cond(nz > 0, extract, lambda cnt: cnt, cnt)

        return lax.fori_loop(0, WORDS // L, do_group, cnt)

    cnt = lax.fori_loop(0, XROWS, do_row, jnp.int32(0))
    cnt_pad = jnp.maximum(((cnt + (2 * B - 1)) // (2 * B)) * (2 * B),
                          2 * B)

    pltpu.sync_copy(src_buf, src_hbm.at[pl.ds(w * CAP_SUB, CAP_SUB)])
    pltpu.sync_copy(dst_buf, dst_hbm.at[pl.ds(w * CAP_SUB, CAP_SUB)])
    cnt_v[...] = jnp.full((L,), 0, jnp.int32) + cnt_pad
    pltpu.sync_copy(cnt_v, cnt_hbm.at[pl.ds(w * L, L)])


def _sigmoid_softplus(rows_d, rows_s, msg, e):
    for c in range(HID // L):
        zf = rows_d[e, pl.ds(c * L, L)] + rows_s[e, pl.ds(c * L, L)]
        zs = (rows_d[e, pl.ds(HID + c * L, L)]
              + rows_s[e, pl.ds(HID + c * L, L)])
        sig = 1.0 / (1.0 + jnp.exp(-zf))
        t = jnp.exp(-jnp.abs(zs))
        acc = jnp.full((L,), _LOG1P[-1], jnp.float32)
        for a in _LOG1P[-2::-1]:
            acc = acc * t + a
        sp = jnp.maximum(zs, 0.0) + acc
        msg[e, pl.ds(c * L, L)] = sig * sp


def _edge_prologue(zeros_hbm, agg_sh):
    sid = lax.axis_index("s")
    pltpu.sync_copy(zeros_hbm.at[pl.ds(sid * RSUB, RSUB)],
                    agg_sh.at[pl.ds(sid * RSUB, RSUB)])
    plsc.subcore_barrier()


def _edge_epilogue(agg_sh, out_hbm):
    plsc.subcore_barrier()
    cid = lax.axis_index("c")
    sid = lax.axis_index("s")
    base = cid * ROWS + sid * RSUB
    pltpu.sync_copy(agg_sh.at[pl.ds(sid * RSUB, RSUB)],
                    out_hbm.at[pl.ds(base, RSUB)])


def _edge_compute(p, rows_d, rows_s, idx_d, sidx, msg, agg_sh, sem):
    @pl.when(p > 0)
    def _():
        pltpu.make_async_copy(msg, agg_sh.at[sidx], sem).wait()
    for i in range(B // L):
        sidx[pl.ds(i * L, L)] = idx_d[pl.ds(i * L, L)]

    @plsc.parallel_loop(0, B, unroll=1)
    def _(e):
        _sigmoid_softplus(rows_d, rows_s, msg, e)
    pltpu.async_copy(msg, agg_sh.at[sidx], sem, add=True)


def _edge_pipeline(tdst_hbm, tsrc_hbm, srce_hbm, dste_hbm, base0, npair,
                   nch, idx_s0, idx_d0, idx_s1, idx_d1, sidx0, sidx1,
                   rows_d0, rows_s0, rows_d1, rows_s1, msg0, msg1, agg_sh,
                   sem1, sem2, sem3, sem4, sem5, sem6):
    pltpu.sync_copy(srce_hbm.at[pl.ds(base0, B)], idx_s0)
    pltpu.sync_copy(dste_hbm.at[pl.ds(base0, B)], idx_d0)
    cp1 = pltpu.async_copy(tdst_hbm.at[idx_d0], rows_d0, sem1)
    cp2 = pltpu.async_copy(tsrc_hbm.at[idx_s0], rows_s0, sem2)

    def pair(p, _):
        b1 = base0 + (2 * p + 1) * B
        pltpu.sync_copy(srce_hbm.at[pl.ds(b1, B)], idx_s1)
        pltpu.sync_copy(dste_hbm.at[pl.ds(b1, B)], idx_d1)
        cp3 = pltpu.async_copy(tdst_hbm.at[idx_d1], rows_d1, sem3)
        cp4 = pltpu.async_copy(tsrc_hbm.at[idx_s1], rows_s1, sem4)
        cp1.wait()
        cp2.wait()
        _edge_compute(p, rows_d0, rows_s0, idx_d0, sidx0, msg0, agg_sh,
                      sem5)

        @pl.when(2 * p + 2 < nch)
        def _():
            b2 = base0 + (2 * p + 2) * B
            pltpu.sync_copy(srce_hbm.at[pl.ds(b2, B)], idx_s0)
            pltpu.sync_copy(dste_hbm.at[pl.ds(b2, B)], idx_d0)
            pltpu.async_copy(tdst_hbm.at[idx_d0], rows_d0, sem1)
            pltpu.async_copy(tsrc_hbm.at[idx_s0], rows_s0, sem2)

        cp3.wait()
        cp4.wait()
        _edge_compute(p, rows_d1, rows_s1, idx_d1, sidx1, msg1, agg_sh,
                      sem6)
        return 0

    lax.fori_loop(0, npair, pair, 0)
    pltpu.make_async_copy(msg0, agg_sh.at[sidx0], sem5).wait()
    pltpu.make_async_copy(msg1, agg_sh.at[sidx1], sem6).wait()


_EDGE_SCRATCH = [
    pltpu.VMEM((B,), jnp.int32),
    pltpu.VMEM((B,), jnp.int32),
    pltpu.VMEM((B,), jnp.int32),
    pltpu.VMEM((B,), jnp.int32),
    pltpu.VMEM((B,), jnp.int32),
    pltpu.VMEM((B,), jnp.int32),
    pltpu.VMEM((B, 2 * HID), jnp.float32),
    pltpu.VMEM((B, 2 * HID), jnp.float32),
    pltpu.VMEM((B, 2 * HID), jnp.float32),
    pltpu.VMEM((B, 2 * HID), jnp.float32),
    pltpu.VMEM((B, HID), jnp.float32),
    pltpu.VMEM((B, HID), jnp.float32),
    pltpu.VMEM_SHARED((ROWS, HID), jnp.float32),
    pltpu.SemaphoreType.DMA,
    pltpu.SemaphoreType.DMA,
    pltpu.SemaphoreType.DMA,
    pltpu.SemaphoreType.DMA,
    pltpu.SemaphoreType.DMA,
    pltpu.SemaphoreType.DMA,
]

_EDGE_OUT = jax.ShapeDtypeStruct((NC * ROWS, HID), jnp.float32)


@functools.lru_cache(maxsize=None)
def _build_cnt_bond():
    return pl.kernel(
        _cnt_bond_body, out_type=_EDGE_OUT, mesh=_vmesh(),
        compiler_params=_SC_PARAMS,
        scratch_types=[
            pltpu.VMEM((B,), jnp.int32),
            pltpu.VMEM((B, HID), jnp.float32),
            pltpu.VMEM_SHARED((ROWS, HID), jnp.float32),
        ])


def _cnt_bond(dst_b, zeros128):
    return _build_cnt_bond()(dst_b, zeros128)


def _cnt_bond_body(dste_hbm, zeros_hbm, out_hbm, idx_d, ones, cnt_sh):
    w = _wid()
    _edge_prologue(zeros_hbm, cnt_sh)

    def fill(e, _):
        for c in range(HID // L):
            ones[e, pl.ds(c * L, L)] = jnp.full((L,), 1.0, jnp.float32)
        return 0
    lax.fori_loop(0, B, fill, 0)

    def chunk(k, _):
        base = w * EPW + k * B
        pltpu.sync_copy(dste_hbm.at[pl.ds(base, B)], idx_d)
        pltpu.sync_copy(ones, cnt_sh.at[idx_d], add=True)
        return 0
    lax.fori_loop(0, NCHUNK, chunk, 0)
    _edge_epilogue(cnt_sh, out_hbm)


@functools.lru_cache(maxsize=None)
def _build_edges_bond():
    return pl.kernel(_edges_bond_body, out_type=_EDGE_OUT, mesh=_vmesh(),
                     compiler_params=_SC_PARAMS,
                     scratch_types=_EDGE_SCRATCH)


def _edges_bond(td, ts, src_b, dst_b, zeros144):
    return _build_edges_bond()(td, ts, src_b, dst_b, zeros144)


def _edges_bond_body(tdst_hbm, tsrc_hbm, srce_hbm, dste_hbm, zeros_hbm,
                     out_hbm, idx_s0, idx_d0, idx_s1, idx_d1, sidx0,
                     sidx1, rows_d0, rows_s0, rows_d1, rows_s1, msg0,
                     msg1, agg_sh, sem1, sem2, sem3, sem4, sem5, sem6):
    w = _wid()
    _edge_prologue(zeros_hbm, agg_sh)
    _edge_pipeline(tdst_hbm, tsrc_hbm, srce_hbm, dste_hbm, w * EPW,
                   NCHUNK // 2, NCHUNK, idx_s0, idx_d0, idx_s1, idx_d1,
                   sidx0, sidx1, rows_d0, rows_s0, rows_d1, rows_s1,
                   msg0, msg1, agg_sh, sem1, sem2, sem3, sem4, sem5, sem6)
    _edge_epilogue(agg_sh, out_hbm)


@functools.lru_cache(maxsize=None)
def _build_edges_rad():
    return pl.kernel(_edges_rad_body, out_type=_EDGE_OUT, mesh=_vmesh(),
                     compiler_params=_SC_PARAMS,
                     scratch_types=_EDGE_SCRATCH
                     + [pltpu.VMEM((L,), jnp.int32)])


def _edges_rad(td, ts, src_r, dst_r, cnt_r, zeros144):
    return _build_edges_rad()(td, ts, src_r, dst_r, cnt_r, zeros144)


def _edges_rad_body(tdst_hbm, tsrc_hbm, srce_hbm, dste_hbm, cnt_hbm,
                    zeros_hbm, out_hbm, idx_s0, idx_d0, idx_s1, idx_d1,
                    sidx0, sidx1, rows_d0, rows_s0, rows_d1, rows_s1,
                    msg0, msg1, agg_sh, sem1, sem2, sem3, sem4, sem5,
                    sem6, cnt_v):
    w = _wid()
    _edge_prologue(zeros_hbm, agg_sh)
    pltpu.sync_copy(cnt_hbm.at[pl.ds(w * L, L)], cnt_v)
    nch = cnt_v[...][0] // B
    _edge_pipeline(tdst_hbm, tsrc_hbm, srce_hbm, dste_hbm, w * CAP_SUB,
                   nch // 2, nch, idx_s0, idx_d0, idx_s1, idx_d1,
                   sidx0, sidx1, rows_d0, rows_s0, rows_d1, rows_s1,
                   msg0, msg1, agg_sh, sem1, sem2, sem3, sem4, sem5, sem6)
    _edge_epilogue(agg_sh, out_hbm)


def kernel(x_prot, v_prot, edge_index, embed_W1, embed_b1, embed_W2, embed_b2,
           bond1_Wf, bond1_bf, bond1_Ws, bond1_bs,
           bond2_Wf, bond2_bf, bond2_Ws, bond2_bs,
           radi1_Wf, radi1_bf, radi1_Ws, radi1_bs,
           radi2_Wf, radi2_bf, radi2_Ws, radi2_bs):
    f32 = jnp.float32
    x_pad = jnp.zeros((ROWS, HID), f32).at[:N].set(x_prot)
    src_b = jnp.concatenate(
        [edge_index[0], jnp.zeros((E_PAD - E,), jnp.int32)])
    dst_b = jnp.concatenate(
        [edge_index[1], jnp.full((E_PAD - E,), DUMP, jnp.int32)])

    pad = jnp.arange(NP_) >= N
    pos = jnp.zeros((NP_, 8), f32).at[:N, :3].set(v_prot.astype(f32) - 50.0)
    big = (100.0 + 10.0 * (jnp.arange(NP_) - N)).astype(f32)
    pos = pos.at[:, 0].set(jnp.where(pad, big, pos[:, 0]))
    maskw, rowsum = _mask_tc(pos, pos.T, jnp.asarray(_PACK))
    maskw_flat = maskw.reshape(-1)
    src_r, dst_r, cnt_r = _extract_sc(maskw_flat)

    zeros128 = jnp.zeros((ROWS, HID), f32)
    cb = _cnt_bond(dst_b, zeros128)
    cnt_bond_col = (cb[:ROWS, :1] + cb[ROWS:, :1])
    cnt_rad_col = rowsum[:ROWS]
    h = _embed_tc(x_pad, embed_W1, embed_b1.reshape(1, HID),
                  embed_W2, embed_b2.reshape(1, HID))

    layers = [
        (bond1_Wf, bond1_bf, bond1_Ws, bond1_bs, True),
        (bond2_Wf, bond2_bf, bond2_Ws, bond2_bs, True),
        (radi1_Wf, radi1_bf, radi1_Ws, radi1_bs, False),
        (radi2_Wf, radi2_bf, radi2_Ws, radi2_bs, False),
    ]
    for Wf, bf, Ws, bs, is_bond in layers:
        wd = jnp.concatenate([Wf[:HID], Ws[:HID]], axis=1)
        wsrc = jnp.concatenate([Wf[HID:], Ws[HID:]], axis=1)
        bd = jnp.concatenate([bf, bs]).reshape(1, 2 * HID)
        td, ts = _tables_tc(h, wd, wsrc, bd)
        if is_bond:
            parts = _edges_bond(td, ts, src_b, dst_b, zeros128)
            cnt_col = cnt_bond_col
        else:
            parts = _edges_rad(td, ts, src_r, dst_r, cnt_r, zeros128)
            cnt_col = cnt_rad_col
        h = _update_tc(h, parts[:ROWS], parts[ROWS:], cnt_col)
    return h[:N]


# speedup vs baseline: 4.6144x; 1.0019x over previous
import functools

import numpy as np
import jax
import jax.numpy as jnp
from jax import lax
from jax.experimental import pallas as pl
from jax.experimental.pallas import tpu as pltpu
from jax.experimental.pallas import tpu_sc as plsc

N = 10000
HID = 128
E = 320000
R2 = 16.0
EDGE_CAP = 131072

NC, NS, L = 2, 16, 16
NW = NC * NS
ROWS = 10112
DUMP = 10000
RSUB = ROWS // NS
B = 32
EPW = 10112
E_PAD = NW * EPW
NCHUNK = EPW // B
CAP_SUB = 16384

NP_ = 10240
WORDS = NP_ // 16
MRB, MCB = 512, 2048
XROWS = NP_ // NW

_LOG1P = (3.5075520e-06, 0.99979246, -0.49697793, 0.31459054,
          -0.18878268, 0.08172681, -0.01720806)

_PACK = ((np.arange(MCB)[:, None] // 16 == np.arange(MCB // 16)[None, :])
         * (1 << (np.arange(MCB) % 16))[:, None]).astype(np.float32)

@functools.lru_cache(maxsize=None)
def _vmesh():
    return plsc.VectorSubcoreMesh(core_axis_name="c", subcore_axis_name="s",
                                  num_cores=NC, num_subcores=NS)


_SC_PARAMS = pltpu.CompilerParams(needs_layout_passes=False)


def _wid():
    return lax.axis_index("s") * NC + lax.axis_index("c")


def _embed_body(x_ref, w1_ref, b1_ref, w2_ref, b2_ref, o_ref):
    t = jnp.maximum(
        jnp.dot(x_ref[...], w1_ref[...], preferred_element_type=jnp.float32)
        + b1_ref[...], 0.0)
    o_ref[...] = (jnp.dot(t, w2_ref[...], preferred_element_type=jnp.float32)
                  + b2_ref[...])


def _embed_tc(x, w1, b1, w2, b2):
    rb = 1264
    return pl.pallas_call(
        _embed_body,
        grid=(ROWS // rb,),
        in_specs=[
            pl.BlockSpec((rb, HID), lambda i: (i, 0)),
            pl.BlockSpec((HID, HID), lambda i: (0, 0)),
            pl.BlockSpec((1, HID), lambda i: (0, 0)),
            pl.BlockSpec((HID, HID), lambda i: (0, 0)),
            pl.BlockSpec((1, HID), lambda i: (0, 0)),
        ],
        out_specs=pl.BlockSpec((rb, HID), lambda i: (i, 0)),
        out_shape=jax.ShapeDtypeStruct((ROWS, HID), jnp.float32),
    )(x, w1, b1, w2, b2)


def _tables_body(h_ref, wd_ref, ws_ref, bd_ref, td_ref, ts_ref):
    h = h_ref[...]
    td_ref[...] = (jnp.dot(h, wd_ref[...], preferred_element_type=jnp.float32)
                   + bd_ref[...])
    ts_ref[...] = jnp.dot(h, ws_ref[...], preferred_element_type=jnp.float32)


def _tables_tc(h, wd, ws, bd):
    rb = 1264
    return pl.pallas_call(
        _tables_body,
        grid=(ROWS // rb,),
        in_specs=[
            pl.BlockSpec((rb, HID), lambda i: (i, 0)),
            pl.BlockSpec((HID, 2 * HID), lambda i: (0, 0)),
            pl.BlockSpec((HID, 2 * HID), lambda i: (0, 0)),
            pl.BlockSpec((1, 2 * HID), lambda i: (0, 0)),
        ],
        out_specs=[pl.BlockSpec((rb, 2 * HID), lambda i: (i, 0))] * 2,
        out_shape=[jax.ShapeDtypeStruct((ROWS, 2 * HID), jnp.float32)] * 2,
    )(h, wd, ws, bd)


def _update_body(h_ref, p0_ref, p1_ref, cnt_ref, o_ref):
    agg = p0_ref[...] + p1_ref[...]
    cnt = jnp.maximum(cnt_ref[...], 1.0)
    o_ref[...] = jnp.maximum(agg / cnt + h_ref[...], 0.0)


def _update_tc(h, p0, p1, cnt):
    rb = 1264
    return pl.pallas_call(
        _update_body,
        grid=(ROWS // rb,),
        in_specs=[
            pl.BlockSpec((rb, HID), lambda i: (i, 0)),
            pl.BlockSpec((rb, HID), lambda i: (i, 0)),
            pl.BlockSpec((rb, HID), lambda i: (i, 0)),
            pl.BlockSpec((rb, 1), lambda i: (i, 0)),
        ],
        out_specs=pl.BlockSpec((rb, HID), lambda i: (i, 0)),
        out_shape=jax.ShapeDtypeStruct((ROWS, HID), jnp.float32),
    )(h, p0, p1, cnt)


def _mask_body(a_ref, bt_ref, pack_ref, o_ref, rs_ref):
    a = a_ref[...]
    bt = bt_ref[...]
    asq = jnp.sum(a * a, axis=1, keepdims=True)
    bsq = jnp.sum(bt * bt, axis=0, keepdims=True)
    d2 = asq + bsq - 2.0 * jnp.dot(a, bt, preferred_element_type=jnp.float32,
                                   precision=lax.Precision.HIGHEST)
    i0 = pl.program_id(0)
    j0 = pl.program_id(1)
    rid = i0 * MRB + lax.broadcasted_iota(jnp.int32, (MRB, MCB), 0)
    cid = j0 * MCB + lax.broadcasted_iota(jnp.int32, (MRB, MCB), 1)
    m = ((d2 < R2) & (rid != cid)).astype(jnp.float32)
    w = jnp.dot(m, pack_ref[...], preferred_element_type=jnp.float32)
    o_ref[...] = w.astype(jnp.int32)

    @pl.when(j0 == 0)
    def _():
        rs_ref[...] = jnp.zeros_like(rs_ref)
    rs_ref[...] += jnp.sum(m, axis=1, keepdims=True)


def _mask_tc(pos_pad, pos_t, pack):
    return pl.pallas_call(
        _mask_body,
        grid=(NP_ // MRB, NP_ // MCB),
        in_specs=[
            pl.BlockSpec((MRB, 8), lambda i, j: (i, 0)),
            pl.BlockSpec((8, MCB), lambda i, j: (0, j)),
            pl.BlockSpec((MCB, MCB // 16), lambda i, j: (0, 0)),
        ],
        out_specs=[pl.BlockSpec((MRB, MCB // 16), lambda i, j: (i, j)),
                   pl.BlockSpec((MRB, 1), lambda i, j: (i, 0))],
        out_shape=[jax.ShapeDtypeStruct((NP_, WORDS), jnp.int32),
                   jax.ShapeDtypeStruct((NP_, 1), jnp.float32)],
    )(pos_pad, pos_t, pack)


@functools.lru_cache(maxsize=None)
def _build_extract():
    return pl.kernel(
        _extract_body,
        out_type=(jax.ShapeDtypeStruct((NW * CAP_SUB,), jnp.int32),
                  jax.ShapeDtypeStruct((NW * CAP_SUB,), jnp.int32),
                  jax.ShapeDtypeStruct((NW * L,), jnp.int32)),
        mesh=_vmesh(),
        compiler_params=_SC_PARAMS,
        scratch_types=[
            pltpu.VMEM((WORDS,), jnp.int32),
            pltpu.VMEM((CAP_SUB,), jnp.int32),
            pltpu.VMEM((CAP_SUB,), jnp.int32),
            pltpu.VMEM((L,), jnp.int32),
        ],
    )


def _extract_sc(maskw_flat):
    return _build_extract()(maskw_flat)


def _extract_body(maskw_hbm, src_hbm, dst_hbm, cnt_hbm,
                  row_v, src_buf, dst_buf, cnt_v):
    w = _wid()

    def fill(k, _):
        src_buf[pl.ds(k * L, L)] = jnp.zeros((L,), jnp.int32)
        dst_buf[pl.ds(k * L, L)] = jnp.full((L,), DUMP, jnp.int32)
        return 0
    lax.fori_loop(0, CAP_SUB // L, fill, 0)

    lane = lax.iota(jnp.int32, L)

    def do_row(i, cnt):
        r = w * XROWS + i
        pltpu.sync_copy(maskw_hbm.at[pl.ds(r * WORDS, WORDS)], row_v)

        def do_group(g, cnt):
            wv = row_v[pl.ds(g * L, L)]
            nz = plsc.all_reduce_population_count(wv != 0)[0]

            def extract(cnt):
                c = cnt
                for b in range(16):
                    mb = ((wv >> b) & 1) != 0
                    jv = (g * L + lane) * 16 + b
                    off = jnp.minimum(c, CAP_SUB - L)
                    plsc.store_compressed(src_buf.at[pl.ds(off, L)], jv,
                                          mask=mb)
                    plsc.store_compressed(dst_buf.at[pl.ds(off, L)],
                                          jnp.full((L,), r, jnp.int32),
                                          mask=mb)
                    pc = plsc.all_reduce_population_count(mb)[0]
                    c = jnp.minimum(c + pc, CAP_SUB - L)
                return c

            return lax.cond(nz > 0, extract, lambda cnt: cnt, cnt)

        return lax.fori_loop(0, WORDS // L, do_group, cnt)

    cnt = lax.fori_loop(0, XROWS, do_row, jnp.int32(0))
    cnt_pad = jnp.maximum(((cnt + (2 * B - 1)) // (2 * B)) * (2 * B),
                          2 * B)

    pltpu.sync_copy(src_buf, src_hbm.at[pl.ds(w * CAP_SUB, CAP_SUB)])
    pltpu.sync_copy(dst_buf, dst_hbm.at[pl.ds(w * CAP_SUB, CAP_SUB)])
    cnt_v[...] = jnp.full((L,), 0, jnp.int32) + cnt_pad
    pltpu.sync_copy(cnt_v, cnt_hbm.at[pl.ds(w * L, L)])


def _sigmoid_softplus(rows_d, rows_s, msg, e):
    for c in range(HID // L):
        zf = rows_d[e, pl.ds(c * L, L)] + rows_s[e, pl.ds(c * L, L)]
        zs = (rows_d[e, pl.ds(HID + c * L, L)]
              + rows_s[e, pl.ds(HID + c * L, L)])
        sig = 1.0 / (1.0 + jnp.exp(-zf))
        t = jnp.exp(-jnp.abs(zs))
        acc = jnp.full((L,), _LOG1P[-1], jnp.float32)
        for a in _LOG1P[-2::-1]:
            acc = acc * t + a
        sp = jnp.maximum(zs, 0.0) + acc
        msg[e, pl.ds(c * L, L)] = sig * sp


def _edge_prologue(zeros_hbm, agg_sh):
    sid = lax.axis_index("s")
    pltpu.sync_copy(zeros_hbm.at[pl.ds(sid * RSUB, RSUB)],
                    agg_sh.at[pl.ds(sid * RSUB, RSUB)])
    plsc.subcore_barrier()


def _edge_epilogue(agg_sh, out_hbm):
    plsc.subcore_barrier()
    cid = lax.axis_index("c")
    sid = lax.axis_index("s")
    base = cid * ROWS + sid * RSUB
    pltpu.sync_copy(agg_sh.at[pl.ds(sid * RSUB, RSUB)],
                    out_hbm.at[pl.ds(base, RSUB)])


def _edge_compute(rows_d, rows_s, idx_d, sidx, msg, agg_sh, sem,
                  wait_other):
    for i in range(B // L):
        sidx[pl.ds(i * L, L)] = idx_d[pl.ds(i * L, L)]

    @plsc.parallel_loop(0, B, unroll=1)
    def _(e):
        _sigmoid_softplus(rows_d, rows_s, msg, e)
    wait_other()
    pltpu.async_copy(msg, agg_sh.at[sidx], sem, add=True)


def _edge_pipeline(tdst_hbm, tsrc_hbm, srce_hbm, dste_hbm, base0, npair,
                   nch, idx_s0, idx_d0, idx_s1, idx_d1, sidx0, sidx1,
                   rows_d0, rows_s0, rows_d1, rows_s1, msg0, msg1, agg_sh,
                   sem1, sem2, sem3, sem4, sem5, sem6):
    pltpu.sync_copy(srce_hbm.at[pl.ds(base0, B)], idx_s0)
    pltpu.sync_copy(dste_hbm.at[pl.ds(base0, B)], idx_d0)
    cp1 = pltpu.async_copy(tdst_hbm.at[idx_d0], rows_d0, sem1)
    cp2 = pltpu.async_copy(tsrc_hbm.at[idx_s0], rows_s0, sem2)

    def pair(p, _):
        b1 = base0 + (2 * p + 1) * B
        pltpu.sync_copy(srce_hbm.at[pl.ds(b1, B)], idx_s1)
        pltpu.sync_copy(dste_hbm.at[pl.ds(b1, B)], idx_d1)
        cp3 = pltpu.async_copy(tdst_hbm.at[idx_d1], rows_d1, sem3)
        cp4 = pltpu.async_copy(tsrc_hbm.at[idx_s1], rows_s1, sem4)
        cp1.wait()
        cp2.wait()

        def wait_prev_scat1():
            @pl.when(p > 0)
            def _():
                pltpu.make_async_copy(msg1, agg_sh.at[sidx1], sem6).wait()

        _edge_compute(rows_d0, rows_s0, idx_d0, sidx0, msg0, agg_sh,
                      sem5, wait_prev_scat1)

        @pl.when(2 * p + 2 < nch)
        def _():
            b2 = base0 + (2 * p + 2) * B
            pltpu.sync_copy(srce_hbm.at[pl.ds(b2, B)], idx_s0)
            pltpu.sync_copy(dste_hbm.at[pl.ds(b2, B)], idx_d0)
            pltpu.async_copy(tdst_hbm.at[idx_d0], rows_d0, sem1)
            pltpu.async_copy(tsrc_hbm.at[idx_s0], rows_s0, sem2)

        cp3.wait()
        cp4.wait()

        def wait_scat0():
            pltpu.make_async_copy(msg0, agg_sh.at[sidx0], sem5).wait()

        _edge_compute(rows_d1, rows_s1, idx_d1, sidx1, msg1, agg_sh,
                      sem6, wait_scat0)
        return 0

    lax.fori_loop(0, npair, pair, 0)
    pltpu.make_async_copy(msg1, agg_sh.at[sidx1], sem6).wait()


_EDGE_SCRATCH = [
    pltpu.VMEM((B,), jnp.int32),
    pltpu.VMEM((B,), jnp.int32),
    pltpu.VMEM((B,), jnp.int32),
    pltpu.VMEM((B,), jnp.int32),
    pltpu.VMEM((B,), jnp.int32),
    pltpu.VMEM((B,), jnp.int32),
    pltpu.VMEM((B, 2 * HID), jnp.float32),
    pltpu.VMEM((B, 2 * HID), jnp.float32),
    pltpu.VMEM((B, 2 * HID), jnp.float32),
    pltpu.VMEM((B, 2 * HID), jnp.float32),
    pltpu.VMEM((B, HID), jnp.float32),
    pltpu.VMEM((B, HID), jnp.float32),
    pltpu.VMEM_SHARED((ROWS, HID), jnp.float32),
    pltpu.SemaphoreType.DMA,
    pltpu.SemaphoreType.DMA,
    pltpu.SemaphoreType.DMA,
    pltpu.SemaphoreType.DMA,
    pltpu.SemaphoreType.DMA,
    pltpu.SemaphoreType.DMA,
]

_EDGE_OUT = jax.ShapeDtypeStruct((NC * ROWS, HID), jnp.float32)


@functools.lru_cache(maxsize=None)
def _build_cnt_bond():
    return pl.kernel(
        _cnt_bond_body, out_type=_EDGE_OUT, mesh=_vmesh(),
        compiler_params=_SC_PARAMS,
        scratch_types=[
            pltpu.VMEM((B,), jnp.int32),
            pltpu.VMEM((B, HID), jnp.float32),
            pltpu.VMEM_SHARED((ROWS, HID), jnp.float32),
        ])


def _cnt_bond(dst_b, zeros128):
    return _build_cnt_bond()(dst_b, zeros128)


def _cnt_bond_body(dste_hbm, zeros_hbm, out_hbm, idx_d, ones, cnt_sh):
    w = _wid()
    _edge_prologue(zeros_hbm, cnt_sh)

    def fill(e, _):
        for c in range(HID // L):
            ones[e, pl.ds(c * L, L)] = jnp.full((L,), 1.0, jnp.float32)
        return 0
    lax.fori_loop(0, B, fill, 0)

    def chunk(k, _):
        base = w * EPW + k * B
        pltpu.sync_copy(dste_hbm.at[pl.ds(base, B)], idx_d)
        pltpu.sync_copy(ones, cnt_sh.at[idx_d], add=True)
        return 0
    lax.fori_loop(0, NCHUNK, chunk, 0)
    _edge_epilogue(cnt_sh, out_hbm)


@functools.lru_cache(maxsize=None)
def _build_edges_bond():
    return pl.kernel(_edges_bond_body, out_type=_EDGE_OUT, mesh=_vmesh(),
                     compiler_params=_SC_PARAMS,
                     scratch_types=_EDGE_SCRATCH)


def _edges_bond(td, ts, src_b, dst_b, zeros144):
    return _build_edges_bond()(td, ts, src_b, dst_b, zeros144)


def _edges_bond_body(tdst_hbm, tsrc_hbm, srce_hbm, dste_hbm, zeros_hbm,
                     out_hbm, idx_s0, idx_d0, idx_s1, idx_d1, sidx0,
                     sidx1, rows_d0, rows_s0, rows_d1, rows_s1, msg0,
                     msg1, agg_sh, sem1, sem2, sem3, sem4, sem5, sem6):
    w = _wid()
    _edge_prologue(zeros_hbm, agg_sh)
    _edge_pipeline(tdst_hbm, tsrc_hbm, srce_hbm, dste_hbm, w * EPW,
                   NCHUNK // 2, NCHUNK, idx_s0, idx_d0, idx_s1, idx_d1,
                   sidx0, sidx1, rows_d0, rows_s0, rows_d1, rows_s1,
                   msg0, msg1, agg_sh, sem1, sem2, sem3, sem4, sem5, sem6)
    _edge_epilogue(agg_sh, out_hbm)


@functools.lru_cache(maxsize=None)
def _build_edges_rad():
    return pl.kernel(_edges_rad_body, out_type=_EDGE_OUT, mesh=_vmesh(),
                     compiler_params=_SC_PARAMS,
                     scratch_types=_EDGE_SCRATCH
                     + [pltpu.VMEM((L,), jnp.int32)])


def _edges_rad(td, ts, src_r, dst_r, cnt_r, zeros144):
    return _build_edges_rad()(td, ts, src_r, dst_r, cnt_r, zeros144)


def _edges_rad_body(tdst_hbm, tsrc_hbm, srce_hbm, dste_hbm, cnt_hbm,
                    zeros_hbm, out_hbm, idx_s0, idx_d0, idx_s1, idx_d1,
                    sidx0, sidx1, rows_d0, rows_s0, rows_d1, rows_s1,
                    msg0, msg1, agg_sh, sem1, sem2, sem3, sem4, sem5,
                    sem6, cnt_v):
    w = _wid()
    _edge_prologue(zeros_hbm, agg_sh)
    pltpu.sync_copy(cnt_hbm.at[pl.ds(w * L, L)], cnt_v)
    nch = cnt_v[...][0] // B
    _edge_pipeline(tdst_hbm, tsrc_hbm, srce_hbm, dste_hbm, w * CAP_SUB,
                   nch // 2, nch, idx_s0, idx_d0, idx_s1, idx_d1,
                   sidx0, sidx1, rows_d0, rows_s0, rows_d1, rows_s1,
                   msg0, msg1, agg_sh, sem1, sem2, sem3, sem4, sem5, sem6)
    _edge_epilogue(agg_sh, out_hbm)


def kernel(x_prot, v_prot, edge_index, embed_W1, embed_b1, embed_W2, embed_b2,
           bond1_Wf, bond1_bf, bond1_Ws, bond1_bs,
           bond2_Wf, bond2_bf, bond2_Ws, bond2_bs,
           radi1_Wf, radi1_bf, radi1_Ws, radi1_bs,
           radi2_Wf, radi2_bf, radi2_Ws, radi2_bs):
    f32 = jnp.float32
    x_pad = jnp.zeros((ROWS, HID), f32).at[:N].set(x_prot)
    src_b = jnp.concatenate(
        [edge_index[0], jnp.zeros((E_PAD - E,), jnp.int32)])
    dst_b = jnp.concatenate(
        [edge_index[1], jnp.full((E_PAD - E,), DUMP, jnp.int32)])

    pad = jnp.arange(NP_) >= N
    pos = jnp.zeros((NP_, 8), f32).at[:N, :3].set(v_prot.astype(f32) - 50.0)
    big = (100.0 + 10.0 * (jnp.arange(NP_) - N)).astype(f32)
    pos = pos.at[:, 0].set(jnp.where(pad, big, pos[:, 0]))
    maskw, rowsum = _mask_tc(pos, pos.T, jnp.asarray(_PACK))
    maskw_flat = maskw.reshape(-1)
    src_r, dst_r, cnt_r = _extract_sc(maskw_flat)

    zeros128 = jnp.zeros((ROWS, HID), f32)
    cb = _cnt_bond(dst_b, zeros128)
    cnt_bond_col = (cb[:ROWS, :1] + cb[ROWS:, :1])
    cnt_rad_col = rowsum[:ROWS]
    h = _embed_tc(x_pad, embed_W1, embed_b1.reshape(1, HID),
                  embed_W2, embed_b2.reshape(1, HID))

    layers = [
        (bond1_Wf, bond1_bf, bond1_Ws, bond1_bs, True),
        (bond2_Wf, bond2_bf, bond2_Ws, bond2_bs, True),
        (radi1_Wf, radi1_bf, radi1_Ws, radi1_bs, False),
        (radi2_Wf, radi2_bf, radi2_Ws, radi2_bs, False),
    ]
    for Wf, bf, Ws, bs, is_bond in layers:
        wd = jnp.concatenate([Wf[:HID], Ws[:HID]], axis=1)
        wsrc = jnp.concatenate([Wf[HID:], Ws[HID:]], axis=1)
        bd = jnp.concatenate([bf, bs]).reshape(1, 2 * HID)
        td, ts = _tables_tc(h, wd, wsrc, bd)
        if is_bond:
            parts = _edges_bond(td, ts, src_b, dst_b, zeros128)
            cnt_col = cnt_bond_col
        else:
            parts = _edges_rad(td, ts, src_r, dst_r, cnt_r, zeros128)
            cnt_col = cnt_rad_col
        h = _update_tc(h, parts[:ROWS], parts[ROWS:], cnt_col)
    return h[:N]
